# baseline (speedup 1.0000x reference)
import functools

import jax
import jax.numpy as jnp
from jax import lax
from jax.experimental import pallas as pl
from jax.experimental.pallas import tpu as pltpu
from jax.experimental.pallas import tpu_sc as plsc

N = 10000
E = 320000
D = 128

NC = 2
NS = 16
NW = NC * NS
EPW = E // NW
K = 125
CH = 40
NPAD = 10240
RPT = NPAD // NS


def _sc_agg_body(with_deg, *refs):
    if with_deg:
        (x_hbm, src_hbm, dst_hbm, p_hbm, dg_hbm,
         src_v, dst_v, rows_v, rows_w, ones_v, zdeg_v,
         agg_sp, deg_sp, sem) = refs
    else:
        (x_hbm, src_hbm, dst_hbm, p_hbm,
         src_v, dst_v, rows_v, rows_w, agg_sp, sem) = refs

    c = lax.axis_index("c")
    s = lax.axis_index("s")
    wid = c * NS + s
    base_r = s * RPT

    def zrow(i, carry):
        rows_v[i // 8, pl.ds((i % 8) * 16, 16)] = jnp.zeros((16,), jnp.float32)
        return carry
    lax.fori_loop(0, K * 8, zrow, 0)
    def zpad(i, carry):
        src_v[CH, pl.ds(i * 16, 16)] = jnp.zeros((16,), jnp.int32)
        return carry
    lax.fori_loop(0, 8, zpad, 0)
    if with_deg:
        def fill16(i, carry):
            ones_v[pl.ds(i * 16, 16)] = jnp.ones((16,), jnp.float32)
            return carry
        lax.fori_loop(0, 8, fill16, 0)
        def zdeg16(i, carry):
            zdeg_v[pl.ds(i * 16, 16)] = jnp.zeros((16,), jnp.float32)
            return carry
        lax.fori_loop(0, RPT // 16, zdeg16, 0)

    nfull = RPT // K
    rem = RPT - nfull * K
    for q in range(nfull):
        pltpu.sync_copy(rows_v, agg_sp.at[pl.ds(base_r + q * K, K)])
    pltpu.sync_copy(rows_v.at[pl.ds(0, rem)],
                    agg_sp.at[pl.ds(base_r + nfull * K, rem)])
    if with_deg:
        pltpu.sync_copy(zdeg_v, deg_sp.at[pl.ds(base_r, RPT)])
    plsc.subcore_barrier()

    pltpu.sync_copy(src_hbm.at[wid], src_v.at[pl.ds(0, CH)])
    pltpu.sync_copy(dst_hbm.at[wid], dst_v)

    def scatter(j, buf):
        pltpu.sync_copy(buf, agg_sp.at[dst_v.at[j]], add=True)
        if with_deg:
            pltpu.sync_copy(ones_v.at[pl.ds(0, K)],
                            deg_sp.at[dst_v.at[j]], add=True)

    pltpu.sync_copy(x_hbm.at[src_v.at[0]], rows_v)
    def step2(t, carry):
        j0 = 2 * t
        d1 = pltpu.async_copy(x_hbm.at[src_v.at[j0 + 1]], rows_w, sem)
        scatter(j0, rows_v)
        d1.wait()
        d2 = pltpu.async_copy(x_hbm.at[src_v.at[j0 + 2]], rows_v, sem)
        scatter(j0 + 1, rows_w)
        d2.wait()
        return carry
    lax.fori_loop(0, CH // 2, step2, 0)

    plsc.subcore_barrier()
    pltpu.sync_copy(agg_sp.at[pl.ds(base_r, RPT)],
                    p_hbm.at[c, pl.ds(base_r, RPT)])
    if with_deg:
        pltpu.sync_copy(deg_sp.at[pl.ds(base_r, RPT)],
                        dg_hbm.at[c, pl.ds(base_r, RPT)])


def _make_sc_agg(with_deg):
    mesh = plsc.VectorSubcoreMesh(core_axis_name="c", subcore_axis_name="s",
                                  num_cores=NC, num_subcores=NS)
    out_type = [jax.ShapeDtypeStruct((NC, NPAD, D), jnp.float32)]
    scratch = [
        pltpu.VMEM((CH + 1, K), jnp.int32),
        pltpu.VMEM((CH, K), jnp.int32),
        pltpu.VMEM((K, D), jnp.float32),
        pltpu.VMEM((K, D), jnp.float32),
    ]
    if with_deg:
        out_type.append(jax.ShapeDtypeStruct((NC, NPAD), jnp.float32))
        scratch += [
            pltpu.VMEM((128,), jnp.float32),
            pltpu.VMEM((RPT,), jnp.float32),
        ]
    scratch.append(pltpu.VMEM_SHARED((NPAD, D), jnp.float32))
    if with_deg:
        scratch.append(pltpu.VMEM_SHARED((NPAD,), jnp.float32))
    scratch.append(pltpu.SemaphoreType.DMA)
    return pl.kernel(
        functools.partial(_sc_agg_body, with_deg),
        out_type=tuple(out_type),
        mesh=mesh,
        scratch_types=scratch,
    )


def _dense_body(relu, x_ref, pa_ref, pb_ref, da_ref, db_ref, ws_ref, wn_ref,
                b_ref, o_ref):
    deg = jnp.maximum(da_ref[0] + da_ref[1] + db_ref[0] + db_ref[1], 1.0)
    agg = (pa_ref[0] + pa_ref[1] + pb_ref[0] + pb_ref[1]) / deg
    out = jnp.dot(x_ref[...], ws_ref[...], preferred_element_type=jnp.float32)
    out = out + jnp.dot(agg, wn_ref[...], preferred_element_type=jnp.float32)
    out = out + b_ref[...]
    nrm = jnp.sqrt(jnp.sum(out * out, axis=-1, keepdims=True))
    out = out / (nrm + 1e-8)
    if relu:
        out = jnp.maximum(out, 0.0)
    o_ref[...] = out


def _dense(x, pa, pb, da, db, w_self, w_neigh, b, relu):
    B = 2000
    return pl.pallas_call(
        functools.partial(_dense_body, relu),
        grid=(N // B,),
        in_specs=[
            pl.BlockSpec((B, D), lambda i: (i, 0)),
            pl.BlockSpec((NC, B, D), lambda i: (0, i, 0)),
            pl.BlockSpec((NC, B, D), lambda i: (0, i, 0)),
            pl.BlockSpec((NC, B, 1), lambda i: (0, i, 0)),
            pl.BlockSpec((NC, B, 1), lambda i: (0, i, 0)),
            pl.BlockSpec((D, D), lambda i: (0, 0)),
            pl.BlockSpec((D, D), lambda i: (0, 0)),
            pl.BlockSpec((1, D), lambda i: (0, 0)),
        ],
        out_specs=pl.BlockSpec((B, D), lambda i: (i, 0)),
        out_shape=jax.ShapeDtypeStruct((N, D), jnp.float32),
    )(x, pa, pb, da, db, w_self, w_neigh, b.reshape(1, D))


def kernel(x, edge_index, W1_self, W1_neigh, b1, W2_self, W2_neigh, b2):
    src = edge_index[0].reshape(2, NW, CH, K)
    dst = edge_index[1].reshape(2, NW, CH, K)
    sc_deg = _make_sc_agg(True)
    sc_nod = _make_sc_agg(False)
    p1a, dga = sc_deg(x, src[0], dst[0])
    p1b, dgb = sc_deg(x, src[1], dst[1])
    dga = dga.reshape(NC, NPAD, 1)
    dgb = dgb.reshape(NC, NPAD, 1)
    h = _dense(x, p1a, p1b, dga, dgb, W1_self, W1_neigh, b1, relu=True)
    (p2a,) = sc_nod(h, src[0], dst[0])
    (p2b,) = sc_nod(h, src[1], dst[1])
    out = _dense(h, p2a, p2b, dga, dgb, W2_self, W2_neigh, b2, relu=False)
    return out


# speedup vs baseline: 2.2750x; 2.2750x over previous
import functools

import jax
import jax.numpy as jnp
from jax import lax
from jax.experimental import pallas as pl
from jax.experimental.pallas import tpu as pltpu
from jax.experimental.pallas import tpu_sc as plsc

N = 10000
E = 320000
D = 128

NC = 2
NS = 16
NW = NC * NS
EPW = E // NW
K = 125
CH = EPW // K
NPAD = 10240
RPT = NPAD // NS


def _sc_agg_body(with_deg, *refs):
    if with_deg:
        (x_hbm, src_hbm, dst_hbm, p_hbm, dg_hbm,
         src_v, dst_v, rows_v, ones_v, zdeg_v, agg_sp, deg_sp) = refs
    else:
        (x_hbm, src_hbm, dst_hbm, p_hbm,
         src_v, dst_v, rows_v, agg_sp) = refs

    c = lax.axis_index("c")
    s = lax.axis_index("s")
    wid = c * NS + s
    base_r = s * RPT

    def zrow(i, carry):
        rows_v[i // 8, pl.ds((i % 8) * 16, 16)] = jnp.zeros((16,), jnp.float32)
        return carry
    lax.fori_loop(0, K * 8, zrow, 0)
    if with_deg:
        def fill16(i, carry):
            ones_v[pl.ds(i * 16, 16)] = jnp.ones((16,), jnp.float32)
            return carry
        lax.fori_loop(0, 8, fill16, 0)
        def zdeg16(i, carry):
            zdeg_v[pl.ds(i * 16, 16)] = jnp.zeros((16,), jnp.float32)
            return carry
        lax.fori_loop(0, RPT // 16, zdeg16, 0)

    nfull = RPT // K
    rem = RPT - nfull * K
    for q in range(nfull):
        pltpu.sync_copy(rows_v, agg_sp.at[pl.ds(base_r + q * K, K)])
    pltpu.sync_copy(rows_v.at[pl.ds(0, rem)],
                    agg_sp.at[pl.ds(base_r + nfull * K, rem)])
    if with_deg:
        pltpu.sync_copy(zdeg_v, deg_sp.at[pl.ds(base_r, RPT)])
    plsc.subcore_barrier()

    pltpu.sync_copy(src_hbm.at[wid], src_v)
    pltpu.sync_copy(dst_hbm.at[wid], dst_v)

    def step(j, carry):
        pltpu.sync_copy(x_hbm.at[src_v.at[j]], rows_v)
        pltpu.sync_copy(rows_v, agg_sp.at[dst_v.at[j]], add=True)
        if with_deg:
            pltpu.sync_copy(ones_v.at[pl.ds(0, K)],
                            deg_sp.at[dst_v.at[j]], add=True)
        return carry
    lax.fori_loop(0, CH, step, 0)

    plsc.subcore_barrier()
    pltpu.sync_copy(agg_sp.at[pl.ds(base_r, RPT)],
                    p_hbm.at[c, pl.ds(base_r, RPT)])
    if with_deg:
        pltpu.sync_copy(deg_sp.at[pl.ds(base_r, RPT)],
                        dg_hbm.at[c, pl.ds(base_r, RPT)])


def _make_sc_agg(with_deg):
    mesh = plsc.VectorSubcoreMesh(core_axis_name="c", subcore_axis_name="s",
                                  num_cores=NC, num_subcores=NS)
    out_type = [jax.ShapeDtypeStruct((NC, NPAD, D), jnp.float32)]
    scratch = [
        pltpu.VMEM((CH, K), jnp.int32),
        pltpu.VMEM((CH, K), jnp.int32),
        pltpu.VMEM((K, D), jnp.float32),
    ]
    if with_deg:
        out_type.append(jax.ShapeDtypeStruct((NC, NPAD), jnp.float32))
        scratch += [
            pltpu.VMEM((128,), jnp.float32),
            pltpu.VMEM((RPT,), jnp.float32),
        ]
    scratch.append(pltpu.VMEM_SHARED((NPAD, D), jnp.float32))
    if with_deg:
        scratch.append(pltpu.VMEM_SHARED((NPAD,), jnp.float32))
    return pl.kernel(
        functools.partial(_sc_agg_body, with_deg),
        out_type=tuple(out_type),
        mesh=mesh,
        scratch_types=scratch,
    )


def _dense_body(relu, x_ref, p_ref, d_ref, ws_ref, wn_ref, b_ref, o_ref):
    deg = jnp.maximum(d_ref[0] + d_ref[1], 1.0)
    agg = (p_ref[0] + p_ref[1]) / deg
    out = jnp.dot(x_ref[...], ws_ref[...], preferred_element_type=jnp.float32)
    out = out + jnp.dot(agg, wn_ref[...], preferred_element_type=jnp.float32)
    out = out + b_ref[...]
    nrm = jnp.sqrt(jnp.sum(out * out, axis=-1, keepdims=True))
    out = out / (nrm + 1e-8)
    if relu:
        out = jnp.maximum(out, 0.0)
    o_ref[...] = out


def _dense(x, p, d, w_self, w_neigh, b, relu):
    B = 2000
    return pl.pallas_call(
        functools.partial(_dense_body, relu),
        grid=(N // B,),
        in_specs=[
            pl.BlockSpec((B, D), lambda i: (i, 0)),
            pl.BlockSpec((NC, B, D), lambda i: (0, i, 0)),
            pl.BlockSpec((NC, B, 1), lambda i: (0, i, 0)),
            pl.BlockSpec((D, D), lambda i: (0, 0)),
            pl.BlockSpec((D, D), lambda i: (0, 0)),
            pl.BlockSpec((1, D), lambda i: (0, 0)),
        ],
        out_specs=pl.BlockSpec((B, D), lambda i: (i, 0)),
        out_shape=jax.ShapeDtypeStruct((N, D), jnp.float32),
    )(x, p, d, w_self, w_neigh, b.reshape(1, D))


def kernel(x, edge_index, W1_self, W1_neigh, b1, W2_self, W2_neigh, b2):
    src = edge_index[0].reshape(NW, CH, K)
    dst = edge_index[1].reshape(NW, CH, K)
    p1, dg = _make_sc_agg(True)(x, src, dst)
    dg = dg.reshape(NC, NPAD, 1)
    h = _dense(x, p1, dg, W1_self, W1_neigh, b1, relu=True)
    (p2,) = _make_sc_agg(False)(h, src, dst)
    out = _dense(h, p2, dg, W2_self, W2_neigh, b2, relu=False)
    return out
